# Initial kernel scaffold; baseline (speedup 1.0000x reference)
import jax
import jax.numpy as jnp
from jax.experimental import pallas as pl


def kernel(x, adj, W1, b1, W2, b2, W3, b3):
    raise NotImplementedError("write your pallas kernel here")


# speedup vs baseline: 1.0328x; 1.0328x over previous
import functools

import jax
import jax.numpy as jnp
from jax.experimental import pallas as pl
from jax.experimental.pallas import tpu as pltpu


def _layer_kernel(adj_ref, h_ref, W_ref, b_ref, o_ref, *, relu):
    g = jnp.dot(adj_ref[...], h_ref[...], preferred_element_type=jnp.float32)
    o = jnp.dot(g, W_ref[...], preferred_element_type=jnp.float32) + b_ref[...]
    if relu:
        o = jnp.maximum(o, 0.0)
    o_ref[...] = o


def _gcn_layer(adj, h, W, b, relu, bm):
    n = adj.shape[0]
    k = h.shape[1]
    d = W.shape[1]
    return pl.pallas_call(
        functools.partial(_layer_kernel, relu=relu),
        grid=(n // bm,),
        in_specs=[
            pl.BlockSpec((bm, n), lambda i: (i, 0)),
            pl.BlockSpec((n, k), lambda i: (0, 0)),
            pl.BlockSpec((k, d), lambda i: (0, 0)),
            pl.BlockSpec((1, d), lambda i: (0, 0)),
        ],
        out_specs=pl.BlockSpec((bm, d), lambda i: (i, 0)),
        out_shape=jax.ShapeDtypeStruct((n, d), jnp.float32),
        compiler_params=pltpu.CompilerParams(
            dimension_semantics=("parallel",),
        ),
    )(adj, h, W, b.reshape(1, -1))


def kernel(x, adj, W1, b1, W2, b2, W3, b3):
    bm = 400
    h = _gcn_layer(adj, x, W1, b1, True, bm)
    h = _gcn_layer(adj, h, W2, b2, True, bm)
    return _gcn_layer(adj, h, W3, b3, False, bm)
